# baseline (speedup 1.0000x reference)
import dataclasses
import functools

import jax
import jax.numpy as jnp
from jax import lax
from jax.experimental import pallas as pl
from jax.experimental.pallas import tpu as pltpu
from jax.experimental.pallas import tpu_sc as plsc

F32 = jnp.float32
BN = (1.0 + 1e-5) ** -0.5

N_NODES = 10000
NE = 160000
NE_PAD = 163840
B_PAIRS = 8192
N_PAD = 10240

_mesh = plsc.VectorSubcoreMesh(core_axis_name="c", subcore_axis_name="s")

_sc_params = pltpu.CompilerParams()
if "needs_layout_passes" in pltpu.CompilerParams.__dataclass_fields__:
    _sc_params = dataclasses.replace(_sc_params, needs_layout_passes=False)


def _dot(a, b):
    return jnp.dot(a.astype(jnp.bfloat16), b.astype(jnp.bfloat16),
                   preferred_element_type=F32)


def _ln(x):
    m = jnp.mean(x, axis=-1, keepdims=True)
    v = jnp.mean((x - m) ** 2, axis=-1, keepdims=True)
    return (x - m) / jnp.sqrt(v + 1e-5)


def _enc_body(x_ref, w0_ref, b0_ref, w1_ref, b1_ref, gw_ref, nodes_ref, x_out_ref):
    z = _dot(x_ref[...], w0_ref[...]) + b0_ref[...]
    a = _ln(BN * jnp.maximum(z, 0.0))
    z = _dot(a, w1_ref[...]) + b1_ref[...]
    b = _ln(BN * jnp.maximum(z, 0.0))
    node = _ln(BN * b)
    nodes_ref[...] = node
    x_out_ref[...] = _dot(node, gw_ref[...])


def _encoder(x, w0t, b0, w1t, b1, gwt, block):
    n, din = x.shape
    dmid = w0t.shape[1]
    grid = (n // block,)
    full = lambda shape: pl.BlockSpec(shape, lambda i: (0, 0))
    return pl.pallas_call(
        _enc_body,
        grid=grid,
        in_specs=[
            pl.BlockSpec((block, din), lambda i: (i, 0)),
            full((din, dmid)),
            full((1, dmid)),
            full((dmid, 256)),
            full((1, 256)),
            full((256, 256)),
        ],
        out_specs=[
            pl.BlockSpec((block, 256), lambda i: (i, 0)),
            pl.BlockSpec((block, 256), lambda i: (i, 0)),
        ],
        out_shape=[
            jax.ShapeDtypeStruct((n, 256), F32),
            jax.ShapeDtypeStruct((n, 256), F32),
        ],
    )(x, w0t, b0, w1t, b1, gwt)


def _prep_body(x_ref, da_ref, db_ref, xs0_ref, xs1_ref, dis_ref):
    deg = da_ref[...] + db_ref[...] + 1.0
    dis = lax.rsqrt(deg)
    xs = x_ref[...] * dis
    xs0_ref[...] = xs[:, :128]
    xs1_ref[...] = xs[:, 128:]
    dis_ref[...] = dis


def _prep(x, dega, degb):
    block = 1000
    grid = (N_NODES // block,)
    return pl.pallas_call(
        _prep_body,
        grid=grid,
        in_specs=[
            pl.BlockSpec((block, 256), lambda i: (i, 0)),
            pl.BlockSpec((block, 1), lambda i: (i, 0)),
            pl.BlockSpec((block, 1), lambda i: (i, 0)),
        ],
        out_specs=[
            pl.BlockSpec((block, 128), lambda i: (i, 0)),
            pl.BlockSpec((block, 128), lambda i: (i, 0)),
            pl.BlockSpec((block, 1), lambda i: (i, 0)),
        ],
        out_shape=[
            jax.ShapeDtypeStruct((N_NODES, 128), F32),
            jax.ShapeDtypeStruct((N_NODES, 128), F32),
            jax.ShapeDtypeStruct((N_NODES, 1), F32),
        ],
    )(x, dega, degb)


def _post_body(a0_ref, a1_ref, dis_ref, gb_ref, out_ref):
    agg = jnp.concatenate([a0_ref[...], a1_ref[...]], axis=1)
    agg = agg * dis_ref[...] + gb_ref[...]
    e = jnp.where(agg > 0.0, agg, jnp.exp(agg) - 1.0)
    out_ref[...] = _ln(BN * e)


def _post(acc0, acc1, dis, gb):
    block = 1000
    grid = (N_NODES // block,)
    return pl.pallas_call(
        _post_body,
        grid=grid,
        in_specs=[
            pl.BlockSpec((block, 128), lambda i: (i, 0)),
            pl.BlockSpec((block, 128), lambda i: (i, 0)),
            pl.BlockSpec((block, 1), lambda i: (i, 0)),
            pl.BlockSpec((1, 256), lambda i: (0, 0)),
        ],
        out_specs=pl.BlockSpec((block, 256), lambda i: (i, 0)),
        out_shape=jax.ShapeDtypeStruct((N_NODES, 256), F32),
    )(acc0, acc1, dis, gb)


def _ban_body(d2_ref, p2_ref, p0_ref, d0_ref, vw_ref, vb_ref, qw_ref, qb_ref,
              hm_ref, hb_ref, pool_ref, attp_ref, w0_ref, b0_ref, w1_ref,
              b1_ref, wo_ref, bo_ref, out_ref):
    v = jnp.maximum(_dot(d2_ref[...], vw_ref[...]) + vb_ref[...], 0.0)
    q = jnp.maximum(_dot(p2_ref[...], qw_ref[...]) + qb_ref[...], 0.0)
    p = v * q
    s = _dot(p, hm_ref[...]) + hb_ref[...]
    ssum = s[:, 0:1] + s[:, 1:2]
    pooled = _dot(p, pool_ref[...])
    logits = ssum * pooled
    ap = attp_ref[...]
    e = jnp.exp(ap - jnp.max(ap))
    a = e / jnp.sum(e)
    pair = (a[0:1, 0:1] * (BN * logits) + a[0:1, 1:2] * p0_ref[...]
            + a[0:1, 2:3] * d0_ref[...])
    pair = BN * jnp.maximum(_dot(pair, w0_ref[...]) + b0_ref[...], 0.0)
    pair = BN * jnp.maximum(_dot(pair, w1_ref[...]) + b1_ref[...], 0.0)
    z = _dot(pair, wo_ref[...]) + bo_ref[...]
    out_ref[...] = 1.0 / (1.0 + jnp.exp(-z))


def _ban(d2, p2, p0, d0, vwt, vb, qwt, qb, hm, hb, pool, attp, w0t, b0, w1t,
         b1, wot, bo):
    block = 1024
    grid = (B_PAIRS // block,)
    rowspec = pl.BlockSpec((block, 256), lambda i: (i, 0))
    full = lambda shape: pl.BlockSpec(shape, lambda i: (0, 0))
    return pl.pallas_call(
        _ban_body,
        grid=grid,
        in_specs=[
            rowspec, rowspec, rowspec, rowspec,
            full((256, 768)), full((1, 768)),
            full((256, 768)), full((1, 768)),
            full((768, 2)), full((1, 2)),
            full((768, 256)), full((1, 3)),
            full((256, 256)), full((1, 256)),
            full((256, 128)), full((1, 128)),
            full((128, 1)), full((1, 1)),
        ],
        out_specs=pl.BlockSpec((block, 1), lambda i: (i, 0)),
        out_shape=jax.ShapeDtypeStruct((B_PAIRS, 1), F32),
    )(d2, p2, p0, d0, vwt, vb, qwt, qb, hm, hb, pool, attp, w0t, b0, w1t, b1,
      wot, bo)


@functools.partial(
    pl.kernel,
    out_type=jax.ShapeDtypeStruct((20480, 16), F32),
    mesh=_mesh,
    scratch_types=[
        pltpu.VMEM((128,), jnp.int32),
        pltpu.VMEM((128, 16), F32),
        pltpu.VMEM_SHARED((10240, 16), F32),
        pltpu.SemaphoreType.DMA,
    ],
)
def _sc_deg(col_hbm, ew16_hbm, zeros_hbm, out_hbm, col_v, val_v, acc_sh, sem):
    c = lax.axis_index("c")
    s = lax.axis_index("s")
    w = c * 16 + s
    pltpu.sync_copy(zeros_hbm.at[pl.ds(s * 640, 640)],
                    acc_sh.at[pl.ds(s * 640, 640)])
    plsc.subcore_barrier()

    @pl.loop(0, 40)
    def _(i):
        pltpu.sync_copy(col_hbm.at[w, i], col_v)
        pltpu.sync_copy(ew16_hbm.at[pl.ds((w * 40 + i) * 128, 128)], val_v)
        pltpu.sync_copy(val_v, acc_sh.at[col_v], add=True)

    plsc.subcore_barrier()
    pltpu.sync_copy(acc_sh.at[pl.ds(s * 640, 640)],
                    out_hbm.at[pl.ds(c * 10240 + s * 640, 640)])


@functools.partial(
    pl.kernel,
    out_type=jax.ShapeDtypeStruct((2 * N_PAD, 128), F32),
    mesh=_mesh,
    scratch_types=[
        pltpu.VMEM((80, 128), jnp.int32),
        pltpu.VMEM((128, 128), F32),
        pltpu.VMEM((128, 128), F32),
        pltpu.VMEM((128,), jnp.int32),
        pltpu.VMEM((128,), jnp.int32),
        pltpu.VMEM((128,), jnp.int32),
        pltpu.VMEM((128,), F32),
        pltpu.VMEM_SHARED((N_PAD, 128), F32),
        pltpu.SemaphoreType.DMA,
    ],
    compiler_params=_sc_params,
)
def _sc_scatter(xs_hbm, row_hbm, col_hbm, ew_hbm, out_hbm, row_v, rw0, rw1,
                rbuf0, rbuf1, cbuf, ewb, acc_sh, gsem):
    c = lax.axis_index("c")
    s = lax.axis_index("s")
    rows = (rw0, rw1)
    rbufs = (rbuf0, rbuf1)

    def _fill_rbuf(ii, b):
        for k in range(8):
            sl = pl.ds(k * 16, 16)
            rbufs[b].at[sl][...] = row_v.at[ii, sl][...]

    def _gissue(b):
        return pltpu.async_copy(xs_hbm.at[rbufs[b]], rows[b], gsem)

    pltpu.sync_copy(row_hbm.at[c, s], row_v)

    @pl.loop(0, 5)
    def _(t):
        r0 = s * 640 + t * 128
        pltpu.sync_copy(xs_hbm.at[pl.ds(c * N_PAD + r0, 128)],
                        acc_sh.at[pl.ds(r0, 128)])

    _fill_rbuf(0, 0)
    _gissue(0).wait()
    plsc.subcore_barrier()

    def _body(ii, b, do_next):
        bn = (b + 1) % 2
        if do_next:
            _fill_rbuf(ii + 1, bn)
            d = _gissue(bn)
        pltpu.sync_copy(ew_hbm.at[s, ii], ewb)
        pltpu.sync_copy(col_hbm.at[s, ii], cbuf)
        z16 = jnp.zeros((16,), jnp.int32)
        rowb = rows[b]

        @pl.loop(0, 128, unroll=4)
        def _(j):
            sv = plsc.load_gather(ewb, [z16 + j])
            for k in range(8):
                sl = (j, pl.ds(k * 16, 16))
                rowb.at[sl][...] = rowb.at[sl][...] * sv

        if do_next:
            d.wait()
        pltpu.sync_copy(rowb, acc_sh.at[cbuf], add=True)

    @pl.loop(0, 78, step=2)
    def _(i):
        for b in range(2):
            _body(i + b, b, True)

    _body(78, 0, True)
    _body(79, 1, False)
    plsc.subcore_barrier()

    @pl.loop(0, 5)
    def _(t):
        r0 = s * 640 + t * 128
        pltpu.sync_copy(acc_sh.at[pl.ds(r0, 128)],
                        out_hbm.at[pl.ds(c * N_PAD + r0, 128)])


@functools.partial(
    pl.kernel,
    out_type=jax.ShapeDtypeStruct((2 * B_PAIRS, 256), F32),
    mesh=_mesh,
    scratch_types=[
        pltpu.VMEM((128,), jnp.int32),
        pltpu.VMEM((128, 256), F32),
        pltpu.SemaphoreType.DMA,
    ],
)
def _sc_gather(table_hbm, idx_hbm, out_hbm, idx_v, rows_v, sem):
    c = lax.axis_index("c")
    s = lax.axis_index("s")
    base = (s * 2 + c) * 512

    @pl.loop(0, 4)
    def _(i):
        off = base + i * 128
        pltpu.sync_copy(idx_hbm.at[pl.ds(off, 128)], idx_v)
        pltpu.async_copy(table_hbm.at[idx_v], rows_v, sem).wait()
        pltpu.sync_copy(rows_v, out_hbm.at[pl.ds(off, 128)])


def kernel(Proteins, Drugs, edge_index, protein_index, drug_index, edge_weight,
           pW0, pb0, pW1, pb1, dW0, db0, dW1, db1, gW, gb, vW, vb, qW, qb,
           h_mat, h_bias, decW0, decb0, decW1, decb1, outW, outb, att_param):
    i32 = jnp.int32
    npad = NE_PAD - NE
    row = jnp.concatenate([edge_index[0], jnp.zeros((npad,), i32)])
    col = jnp.concatenate([edge_index[1], jnp.zeros((npad,), i32)])
    ew = jnp.concatenate([edge_weight, jnp.zeros((npad,), F32)])
    ew16 = jnp.broadcast_to(ew[:, None], (NE_PAD, 16))
    zeros16 = jnp.zeros((10240, 16), F32)

    deg_flat = _sc_deg(col.reshape(32, 40, 128), ew16, zeros16)
    dega = deg_flat[:N_NODES, 0:1]
    degb = deg_flat[10240:10240 + N_NODES, 0:1]

    nodes_p, x_p = _encoder(Proteins, pW0.T, pb0[None, :], pW1.T, pb1[None, :],
                            gW.T, 1000)
    nodes_d, x_d = _encoder(Drugs, dW0.T, db0[None, :], dW1.T, db1[None, :],
                            gW.T, 1000)
    nodes = jnp.concatenate([nodes_p, nodes_d], axis=0)
    x = jnp.concatenate([x_p, x_d], axis=0)

    idx_all = jnp.concatenate([protein_index, drug_index])
    g0 = _sc_gather(nodes, idx_all)
    p0, d0 = g0[:B_PAIRS], g0[B_PAIRS:]

    xs0, xs1, dis = _prep(x, dega, degb)
    zpad = jnp.zeros((N_PAD - N_NODES, 128), F32)
    xs_flat = jnp.concatenate([xs0, zpad, xs1, zpad], axis=0)
    rowp = jnp.stack([row, row + N_PAD]).reshape(2, 16, 80, 128)
    acc_flat = _sc_scatter(xs_flat, rowp, col.reshape(16, 80, 128),
                           ew.reshape(16, 80, 128))
    nodes2 = _post(acc_flat[:N_NODES], acc_flat[N_PAD:N_PAD + N_NODES], dis,
                   gb[None, :])

    g2 = _sc_gather(nodes2, idx_all)
    p2, d2 = g2[:B_PAIRS], g2[B_PAIRS:]

    hm = h_mat[0, :, 0, :].T
    hb = h_bias[0, :, 0, 0][None, :]
    pool = (jnp.arange(768)[:, None] // 3 == jnp.arange(256)[None, :]).astype(F32)
    attp = att_param[:, 0, 0][None, :]

    out = _ban(d2, p2, p0, d0, vW.T, vb[None, :], qW.T, qb[None, :], hm, hb,
               pool, attp, decW0.T, decb0[None, :], decW1.T, decb1[None, :],
               outW.T, outb[None, :])
    return out.reshape(-1)


# speedup vs baseline: 1.0020x; 1.0020x over previous
import dataclasses
import functools

import jax
import jax.numpy as jnp
from jax import lax
from jax.experimental import pallas as pl
from jax.experimental.pallas import tpu as pltpu
from jax.experimental.pallas import tpu_sc as plsc

F32 = jnp.float32
BN = (1.0 + 1e-5) ** -0.5

N_NODES = 10000
NE = 160000
NE_PAD = 163840
B_PAIRS = 8192
N_PAD = 10240

_mesh = plsc.VectorSubcoreMesh(core_axis_name="c", subcore_axis_name="s")

_sc_params = pltpu.CompilerParams()
if "needs_layout_passes" in pltpu.CompilerParams.__dataclass_fields__:
    _sc_params = dataclasses.replace(_sc_params, needs_layout_passes=False)


def _dot(a, b):
    return jnp.dot(a, b, preferred_element_type=F32)


def _ln(x):
    m = jnp.mean(x, axis=-1, keepdims=True)
    v = jnp.mean((x - m) ** 2, axis=-1, keepdims=True)
    return (x - m) / jnp.sqrt(v + 1e-5)


def _enc_body(x_ref, w0_ref, b0_ref, w1_ref, b1_ref, gw_ref, nodes_ref, x_out_ref):
    z = _dot(x_ref[...], w0_ref[...]) + b0_ref[...]
    a = _ln(BN * jnp.maximum(z, 0.0))
    z = _dot(a, w1_ref[...]) + b1_ref[...]
    b = _ln(BN * jnp.maximum(z, 0.0))
    node = _ln(BN * b)
    nodes_ref[...] = node
    x_out_ref[...] = _dot(node, gw_ref[...])


def _encoder(x, w0t, b0, w1t, b1, gwt, block):
    n, din = x.shape
    dmid = w0t.shape[1]
    grid = (n // block,)
    full = lambda shape: pl.BlockSpec(shape, lambda i: (0, 0))
    return pl.pallas_call(
        _enc_body,
        grid=grid,
        in_specs=[
            pl.BlockSpec((block, din), lambda i: (i, 0)),
            full((din, dmid)),
            full((1, dmid)),
            full((dmid, 256)),
            full((1, 256)),
            full((256, 256)),
        ],
        out_specs=[
            pl.BlockSpec((block, 256), lambda i: (i, 0)),
            pl.BlockSpec((block, 256), lambda i: (i, 0)),
        ],
        out_shape=[
            jax.ShapeDtypeStruct((n, 256), F32),
            jax.ShapeDtypeStruct((n, 256), F32),
        ],
    )(x, w0t, b0, w1t, b1, gwt)


def _prep_body(x_ref, da_ref, db_ref, xs0_ref, xs1_ref, dis_ref):
    deg = da_ref[...] + db_ref[...] + 1.0
    dis = lax.rsqrt(deg)
    xs = x_ref[...] * dis
    xs0_ref[...] = xs[:, :128]
    xs1_ref[...] = xs[:, 128:]
    dis_ref[...] = dis


def _prep(x, dega, degb):
    block = 1000
    grid = (N_NODES // block,)
    return pl.pallas_call(
        _prep_body,
        grid=grid,
        in_specs=[
            pl.BlockSpec((block, 256), lambda i: (i, 0)),
            pl.BlockSpec((block, 1), lambda i: (i, 0)),
            pl.BlockSpec((block, 1), lambda i: (i, 0)),
        ],
        out_specs=[
            pl.BlockSpec((block, 128), lambda i: (i, 0)),
            pl.BlockSpec((block, 128), lambda i: (i, 0)),
            pl.BlockSpec((block, 1), lambda i: (i, 0)),
        ],
        out_shape=[
            jax.ShapeDtypeStruct((N_NODES, 128), F32),
            jax.ShapeDtypeStruct((N_NODES, 128), F32),
            jax.ShapeDtypeStruct((N_NODES, 1), F32),
        ],
    )(x, dega, degb)


def _post_body(a0_ref, a1_ref, dis_ref, gb_ref, out_ref):
    agg = jnp.concatenate([a0_ref[...], a1_ref[...]], axis=1)
    agg = agg * dis_ref[...] + gb_ref[...]
    e = jnp.where(agg > 0.0, agg, jnp.exp(agg) - 1.0)
    out_ref[...] = _ln(BN * e)


def _post(acc0, acc1, dis, gb):
    block = 1000
    grid = (N_NODES // block,)
    return pl.pallas_call(
        _post_body,
        grid=grid,
        in_specs=[
            pl.BlockSpec((block, 128), lambda i: (i, 0)),
            pl.BlockSpec((block, 128), lambda i: (i, 0)),
            pl.BlockSpec((block, 1), lambda i: (i, 0)),
            pl.BlockSpec((1, 256), lambda i: (0, 0)),
        ],
        out_specs=pl.BlockSpec((block, 256), lambda i: (i, 0)),
        out_shape=jax.ShapeDtypeStruct((N_NODES, 256), F32),
    )(acc0, acc1, dis, gb)


def _ban_body(d2_ref, p2_ref, p0_ref, d0_ref, vw_ref, vb_ref, qw_ref, qb_ref,
              hm_ref, hb_ref, pool_ref, attp_ref, w0_ref, b0_ref, w1_ref,
              b1_ref, wo_ref, bo_ref, out_ref):
    v = jnp.maximum(_dot(d2_ref[...], vw_ref[...]) + vb_ref[...], 0.0)
    q = jnp.maximum(_dot(p2_ref[...], qw_ref[...]) + qb_ref[...], 0.0)
    p = v * q
    s = _dot(p, hm_ref[...]) + hb_ref[...]
    ssum = s[:, 0:1] + s[:, 1:2]
    pooled = _dot(p, pool_ref[...])
    logits = ssum * pooled
    ap = attp_ref[...]
    e = jnp.exp(ap - jnp.max(ap))
    a = e / jnp.sum(e)
    pair = (a[0:1, 0:1] * (BN * logits) + a[0:1, 1:2] * p0_ref[...]
            + a[0:1, 2:3] * d0_ref[...])
    pair = BN * jnp.maximum(_dot(pair, w0_ref[...]) + b0_ref[...], 0.0)
    pair = BN * jnp.maximum(_dot(pair, w1_ref[...]) + b1_ref[...], 0.0)
    z = _dot(pair, wo_ref[...]) + bo_ref[...]
    out_ref[...] = 1.0 / (1.0 + jnp.exp(-z))


def _ban(d2, p2, p0, d0, vwt, vb, qwt, qb, hm, hb, pool, attp, w0t, b0, w1t,
         b1, wot, bo):
    block = 1024
    grid = (B_PAIRS // block,)
    rowspec = pl.BlockSpec((block, 256), lambda i: (i, 0))
    full = lambda shape: pl.BlockSpec(shape, lambda i: (0, 0))
    return pl.pallas_call(
        _ban_body,
        grid=grid,
        in_specs=[
            rowspec, rowspec, rowspec, rowspec,
            full((256, 768)), full((1, 768)),
            full((256, 768)), full((1, 768)),
            full((768, 2)), full((1, 2)),
            full((768, 256)), full((1, 3)),
            full((256, 256)), full((1, 256)),
            full((256, 128)), full((1, 128)),
            full((128, 1)), full((1, 1)),
        ],
        out_specs=pl.BlockSpec((block, 1), lambda i: (i, 0)),
        out_shape=jax.ShapeDtypeStruct((B_PAIRS, 1), F32),
    )(d2, p2, p0, d0, vwt, vb, qwt, qb, hm, hb, pool, attp, w0t, b0, w1t, b1,
      wot, bo)


@functools.partial(
    pl.kernel,
    out_type=jax.ShapeDtypeStruct((20480, 16), F32),
    mesh=_mesh,
    scratch_types=[
        pltpu.VMEM((128,), jnp.int32),
        pltpu.VMEM((128, 16), F32),
        pltpu.VMEM_SHARED((10240, 16), F32),
        pltpu.SemaphoreType.DMA,
    ],
)
def _sc_deg(col_hbm, ew16_hbm, zeros_hbm, out_hbm, col_v, val_v, acc_sh, sem):
    c = lax.axis_index("c")
    s = lax.axis_index("s")
    w = c * 16 + s
    pltpu.sync_copy(zeros_hbm.at[pl.ds(s * 640, 640)],
                    acc_sh.at[pl.ds(s * 640, 640)])
    plsc.subcore_barrier()

    @pl.loop(0, 40)
    def _(i):
        pltpu.sync_copy(col_hbm.at[w, i], col_v)
        pltpu.sync_copy(ew16_hbm.at[pl.ds((w * 40 + i) * 128, 128)], val_v)
        pltpu.sync_copy(val_v, acc_sh.at[col_v], add=True)

    plsc.subcore_barrier()
    pltpu.sync_copy(acc_sh.at[pl.ds(s * 640, 640)],
                    out_hbm.at[pl.ds(c * 10240 + s * 640, 640)])


@functools.partial(
    pl.kernel,
    out_type=jax.ShapeDtypeStruct((2 * N_PAD, 128), F32),
    mesh=_mesh,
    scratch_types=[
        pltpu.VMEM((80, 128), jnp.int32),
        pltpu.VMEM((128, 128), F32),
        pltpu.VMEM((128, 128), F32),
        pltpu.VMEM((128,), jnp.int32),
        pltpu.VMEM((128,), jnp.int32),
        pltpu.VMEM((128,), jnp.int32),
        pltpu.VMEM((128,), F32),
        pltpu.VMEM_SHARED((N_PAD, 128), F32),
        pltpu.SemaphoreType.DMA,
    ],
    compiler_params=_sc_params,
)
def _sc_scatter(xs_hbm, row_hbm, col_hbm, ew_hbm, out_hbm, row_v, rw0, rw1,
                rbuf0, rbuf1, cbuf, ewb, acc_sh, gsem):
    c = lax.axis_index("c")
    s = lax.axis_index("s")
    rows = (rw0, rw1)
    rbufs = (rbuf0, rbuf1)

    def _fill_rbuf(ii, b):
        for k in range(8):
            sl = pl.ds(k * 16, 16)
            rbufs[b].at[sl][...] = row_v.at[ii, sl][...]

    def _gissue(b):
        return pltpu.async_copy(xs_hbm.at[rbufs[b]], rows[b], gsem)

    pltpu.sync_copy(row_hbm.at[c, s], row_v)

    @pl.loop(0, 5)
    def _(t):
        r0 = s * 640 + t * 128
        pltpu.sync_copy(xs_hbm.at[pl.ds(c * N_PAD + r0, 128)],
                        acc_sh.at[pl.ds(r0, 128)])

    _fill_rbuf(0, 0)
    _gissue(0).wait()
    plsc.subcore_barrier()

    def _body(ii, b, do_next):
        bn = (b + 1) % 2
        if do_next:
            _fill_rbuf(ii + 1, bn)
            d = _gissue(bn)
        pltpu.sync_copy(ew_hbm.at[s, ii], ewb)
        pltpu.sync_copy(col_hbm.at[s, ii], cbuf)
        z16 = jnp.zeros((16,), jnp.int32)
        rowb = rows[b]

        @pl.loop(0, 128, unroll=4)
        def _(j):
            sv = plsc.load_gather(ewb, [z16 + j])
            for k in range(8):
                sl = (j, pl.ds(k * 16, 16))
                rowb.at[sl][...] = rowb.at[sl][...] * sv

        if do_next:
            d.wait()
        pltpu.sync_copy(rowb, acc_sh.at[cbuf], add=True)

    @pl.loop(0, 78, step=2)
    def _(i):
        for b in range(2):
            _body(i + b, b, True)

    _body(78, 0, True)
    _body(79, 1, False)
    plsc.subcore_barrier()

    @pl.loop(0, 5)
    def _(t):
        r0 = s * 640 + t * 128
        pltpu.sync_copy(acc_sh.at[pl.ds(r0, 128)],
                        out_hbm.at[pl.ds(c * N_PAD + r0, 128)])


@functools.partial(
    pl.kernel,
    out_type=jax.ShapeDtypeStruct((2 * B_PAIRS, 256), F32),
    mesh=_mesh,
    scratch_types=[
        pltpu.VMEM((128,), jnp.int32),
        pltpu.VMEM((128, 256), F32),
        pltpu.SemaphoreType.DMA,
    ],
)
def _sc_gather(table_hbm, idx_hbm, out_hbm, idx_v, rows_v, sem):
    c = lax.axis_index("c")
    s = lax.axis_index("s")
    base = (s * 2 + c) * 512

    @pl.loop(0, 4)
    def _(i):
        off = base + i * 128
        pltpu.sync_copy(idx_hbm.at[pl.ds(off, 128)], idx_v)
        pltpu.async_copy(table_hbm.at[idx_v], rows_v, sem).wait()
        pltpu.sync_copy(rows_v, out_hbm.at[pl.ds(off, 128)])


def kernel(Proteins, Drugs, edge_index, protein_index, drug_index, edge_weight,
           pW0, pb0, pW1, pb1, dW0, db0, dW1, db1, gW, gb, vW, vb, qW, qb,
           h_mat, h_bias, decW0, decb0, decW1, decb1, outW, outb, att_param):
    i32 = jnp.int32
    npad = NE_PAD - NE
    row = jnp.concatenate([edge_index[0], jnp.zeros((npad,), i32)])
    col = jnp.concatenate([edge_index[1], jnp.zeros((npad,), i32)])
    ew = jnp.concatenate([edge_weight, jnp.zeros((npad,), F32)])
    ew16 = jnp.broadcast_to(ew[:, None], (NE_PAD, 16))
    zeros16 = jnp.zeros((10240, 16), F32)

    deg_flat = _sc_deg(col.reshape(32, 40, 128), ew16, zeros16)
    dega = deg_flat[:N_NODES, 0:1]
    degb = deg_flat[10240:10240 + N_NODES, 0:1]

    nodes_p, x_p = _encoder(Proteins, pW0.T, pb0[None, :], pW1.T, pb1[None, :],
                            gW.T, 1000)
    nodes_d, x_d = _encoder(Drugs, dW0.T, db0[None, :], dW1.T, db1[None, :],
                            gW.T, 1000)
    nodes = jnp.concatenate([nodes_p, nodes_d], axis=0)
    x = jnp.concatenate([x_p, x_d], axis=0)

    idx_all = jnp.concatenate([protein_index, drug_index])
    g0 = _sc_gather(nodes, idx_all)
    p0, d0 = g0[:B_PAIRS], g0[B_PAIRS:]

    xs0, xs1, dis = _prep(x, dega, degb)
    zpad = jnp.zeros((N_PAD - N_NODES, 128), F32)
    xs_flat = jnp.concatenate([xs0, zpad, xs1, zpad], axis=0)
    rowp = jnp.stack([row, row + N_PAD]).reshape(2, 16, 80, 128)
    acc_flat = _sc_scatter(xs_flat, rowp, col.reshape(16, 80, 128),
                           ew.reshape(16, 80, 128))
    nodes2 = _post(acc_flat[:N_NODES], acc_flat[N_PAD:N_PAD + N_NODES], dis,
                   gb[None, :])

    g2 = _sc_gather(nodes2, idx_all)
    p2, d2 = g2[:B_PAIRS], g2[B_PAIRS:]

    hm = h_mat[0, :, 0, :].T
    hb = h_bias[0, :, 0, 0][None, :]
    pool = (jnp.arange(768)[:, None] // 3 == jnp.arange(256)[None, :]).astype(F32)
    attp = att_param[:, 0, 0][None, :]

    out = _ban(d2, p2, p0, d0, vW.T, vb[None, :], qW.T, qb[None, :], hm, hb,
               pool, attp, decW0.T, decb0[None, :], decW1.T, decb1[None, :],
               outW.T, outb[None, :])
    return out.reshape(-1)


# speedup vs baseline: 1.0675x; 1.0653x over previous
import dataclasses
import functools

import jax
import jax.numpy as jnp
from jax import lax
from jax.experimental import pallas as pl
from jax.experimental.pallas import tpu as pltpu
from jax.experimental.pallas import tpu_sc as plsc

F32 = jnp.float32
BN = (1.0 + 1e-5) ** -0.5

N_NODES = 10000
NE = 160000
NE_PAD = 163840
B_PAIRS = 8192
N_PAD = 10240

_mesh = plsc.VectorSubcoreMesh(core_axis_name="c", subcore_axis_name="s")

_sc_params = pltpu.CompilerParams()
if "needs_layout_passes" in pltpu.CompilerParams.__dataclass_fields__:
    _sc_params = dataclasses.replace(_sc_params, needs_layout_passes=False)


def _dot(a, b):
    return jnp.dot(a, b, preferred_element_type=F32)


def _ln(x):
    m = jnp.mean(x, axis=-1, keepdims=True)
    v = jnp.mean((x - m) ** 2, axis=-1, keepdims=True)
    return (x - m) / jnp.sqrt(v + 1e-5)


def _enc_body(x_ref, w0_ref, b0_ref, w1_ref, b1_ref, gw_ref, nodes_ref, x_out_ref):
    z = _dot(x_ref[...], w0_ref[...]) + b0_ref[...]
    a = _ln(BN * jnp.maximum(z, 0.0))
    z = _dot(a, w1_ref[...]) + b1_ref[...]
    b = _ln(BN * jnp.maximum(z, 0.0))
    node = _ln(BN * b)
    nodes_ref[...] = node
    x_out_ref[...] = _dot(node, gw_ref[...])


def _encoder(x, w0t, b0, w1t, b1, gwt, block):
    n, din = x.shape
    dmid = w0t.shape[1]
    grid = (n // block,)
    full = lambda shape: pl.BlockSpec(shape, lambda i: (0, 0))
    return pl.pallas_call(
        _enc_body,
        grid=grid,
        in_specs=[
            pl.BlockSpec((block, din), lambda i: (i, 0)),
            full((din, dmid)),
            full((1, dmid)),
            full((dmid, 256)),
            full((1, 256)),
            full((256, 256)),
        ],
        out_specs=[
            pl.BlockSpec((block, 256), lambda i: (i, 0)),
            pl.BlockSpec((block, 256), lambda i: (i, 0)),
        ],
        out_shape=[
            jax.ShapeDtypeStruct((n, 256), F32),
            jax.ShapeDtypeStruct((n, 256), F32),
        ],
    )(x, w0t, b0, w1t, b1, gwt)


def _prep_body(x_ref, da_ref, db_ref, xs0_ref, xs1_ref, dis_ref):
    deg = da_ref[...] + db_ref[...] + 1.0
    dis = lax.rsqrt(deg)
    xs = x_ref[...] * dis
    xs0_ref[...] = xs[:, :128]
    xs1_ref[...] = xs[:, 128:]
    dis_ref[...] = dis


def _prep(x, dega, degb):
    block = 1000
    grid = (N_NODES // block,)
    return pl.pallas_call(
        _prep_body,
        grid=grid,
        in_specs=[
            pl.BlockSpec((block, 256), lambda i: (i, 0)),
            pl.BlockSpec((block, 1), lambda i: (i, 0)),
            pl.BlockSpec((block, 1), lambda i: (i, 0)),
        ],
        out_specs=[
            pl.BlockSpec((block, 128), lambda i: (i, 0)),
            pl.BlockSpec((block, 128), lambda i: (i, 0)),
            pl.BlockSpec((block, 1), lambda i: (i, 0)),
        ],
        out_shape=[
            jax.ShapeDtypeStruct((N_NODES, 128), F32),
            jax.ShapeDtypeStruct((N_NODES, 128), F32),
            jax.ShapeDtypeStruct((N_NODES, 1), F32),
        ],
    )(x, dega, degb)


def _post_body(a0_ref, a1_ref, dis_ref, gb_ref, out_ref):
    agg = jnp.concatenate([a0_ref[...], a1_ref[...]], axis=1)
    agg = agg * dis_ref[...] + gb_ref[...]
    e = jnp.where(agg > 0.0, agg, jnp.exp(agg) - 1.0)
    out_ref[...] = _ln(BN * e)


def _post(acc0, acc1, dis, gb):
    block = 1000
    grid = (N_NODES // block,)
    return pl.pallas_call(
        _post_body,
        grid=grid,
        in_specs=[
            pl.BlockSpec((block, 128), lambda i: (i, 0)),
            pl.BlockSpec((block, 128), lambda i: (i, 0)),
            pl.BlockSpec((block, 1), lambda i: (i, 0)),
            pl.BlockSpec((1, 256), lambda i: (0, 0)),
        ],
        out_specs=pl.BlockSpec((block, 256), lambda i: (i, 0)),
        out_shape=jax.ShapeDtypeStruct((N_NODES, 256), F32),
    )(acc0, acc1, dis, gb)


def _ban_body(d2_ref, p2_ref, p0_ref, d0_ref, vw_ref, vb_ref, qw_ref, qb_ref,
              hm_ref, hb_ref, pool_ref, attp_ref, w0_ref, b0_ref, w1_ref,
              b1_ref, wo_ref, bo_ref, out_ref):
    v = jnp.maximum(_dot(d2_ref[...], vw_ref[...]) + vb_ref[...], 0.0)
    q = jnp.maximum(_dot(p2_ref[...], qw_ref[...]) + qb_ref[...], 0.0)
    p = v * q
    s = _dot(p, hm_ref[...]) + hb_ref[...]
    ssum = s[:, 0:1] + s[:, 1:2]
    pooled = _dot(p, pool_ref[...])
    logits = ssum * pooled
    ap = attp_ref[...]
    e = jnp.exp(ap - jnp.max(ap))
    a = e / jnp.sum(e)
    pair = (a[0:1, 0:1] * (BN * logits) + a[0:1, 1:2] * p0_ref[...]
            + a[0:1, 2:3] * d0_ref[...])
    pair = BN * jnp.maximum(_dot(pair, w0_ref[...]) + b0_ref[...], 0.0)
    pair = BN * jnp.maximum(_dot(pair, w1_ref[...]) + b1_ref[...], 0.0)
    z = _dot(pair, wo_ref[...]) + bo_ref[...]
    out_ref[...] = 1.0 / (1.0 + jnp.exp(-z))


def _ban(d2, p2, p0, d0, vwt, vb, qwt, qb, hm, hb, pool, attp, w0t, b0, w1t,
         b1, wot, bo):
    block = 1024
    grid = (B_PAIRS // block,)
    rowspec = pl.BlockSpec((block, 256), lambda i: (i, 0))
    full = lambda shape: pl.BlockSpec(shape, lambda i: (0, 0))
    return pl.pallas_call(
        _ban_body,
        grid=grid,
        in_specs=[
            rowspec, rowspec, rowspec, rowspec,
            full((256, 768)), full((1, 768)),
            full((256, 768)), full((1, 768)),
            full((768, 2)), full((1, 2)),
            full((768, 256)), full((1, 3)),
            full((256, 256)), full((1, 256)),
            full((256, 128)), full((1, 128)),
            full((128, 1)), full((1, 1)),
        ],
        out_specs=pl.BlockSpec((block, 1), lambda i: (i, 0)),
        out_shape=jax.ShapeDtypeStruct((B_PAIRS, 1), F32),
    )(d2, p2, p0, d0, vwt, vb, qwt, qb, hm, hb, pool, attp, w0t, b0, w1t, b1,
      wot, bo)


@functools.partial(
    pl.kernel,
    out_type=jax.ShapeDtypeStruct((20480, 16), F32),
    mesh=_mesh,
    scratch_types=[
        pltpu.VMEM((128,), jnp.int32),
        pltpu.VMEM((128,), jnp.int32),
        pltpu.VMEM((128, 16), F32),
        pltpu.VMEM((128, 16), F32),
        pltpu.VMEM_SHARED((10240, 16), F32),
        pltpu.SemaphoreType.DMA,
    ],
)
def _sc_deg(col_hbm, ew16_hbm, zeros_hbm, out_hbm, cv0, cv1, vv0, vv1,
            acc_sh, dsem):
    c = lax.axis_index("c")
    s = lax.axis_index("s")
    cols = (cv0, cv1)
    vals = (vv0, vv1)
    w = c * 16 + s
    pltpu.sync_copy(zeros_hbm.at[pl.ds(s * 640, 640)],
                    acc_sh.at[pl.ds(s * 640, 640)])

    def _body(i, b, do_next):
        bn = (b + 1) % 2
        if do_next:
            d1 = pltpu.async_copy(col_hbm.at[w, i + 1], cols[bn], dsem)
            d2 = pltpu.async_copy(
                ew16_hbm.at[pl.ds((w * 40 + i + 1) * 128, 128)], vals[bn],
                dsem)
        pltpu.sync_copy(vals[b], acc_sh.at[cols[b]], add=True)
        if do_next:
            d1.wait()
            d2.wait()

    pltpu.sync_copy(col_hbm.at[w, 0], cols[0])
    pltpu.sync_copy(ew16_hbm.at[pl.ds(w * 40 * 128, 128)], vals[0])
    plsc.subcore_barrier()

    @pl.loop(0, 38, step=2)
    def _(i):
        for b in range(2):
            _body(i + b, b, True)

    _body(38, 0, True)
    _body(39, 1, False)
    plsc.subcore_barrier()
    pltpu.sync_copy(acc_sh.at[pl.ds(s * 640, 640)],
                    out_hbm.at[pl.ds(c * 10240 + s * 640, 640)])


@functools.partial(
    pl.kernel,
    out_type=jax.ShapeDtypeStruct((2 * N_PAD, 128), F32),
    mesh=_mesh,
    scratch_types=[
        pltpu.VMEM((80, 128), jnp.int32),
        pltpu.VMEM((128, 128), F32),
        pltpu.VMEM((128, 128), F32),
        pltpu.VMEM((128,), jnp.int32),
        pltpu.VMEM((128,), jnp.int32),
        pltpu.VMEM((128,), jnp.int32),
        pltpu.VMEM((128,), F32),
        pltpu.VMEM_SHARED((N_PAD, 128), F32),
        pltpu.SemaphoreType.DMA,
        pltpu.SemaphoreType.DMA,
    ],
    compiler_params=_sc_params,
)
def _sc_scatter(xs_hbm, row_hbm, col_hbm, ew_hbm, out_hbm, row_v, rw0, rw1,
                rbuf0, rbuf1, cbuf, ewb, acc_sh, gsem, dsem):
    c = lax.axis_index("c")
    s = lax.axis_index("s")
    rows = (rw0, rw1)
    rbufs = (rbuf0, rbuf1)

    def _fill_rbuf(ii, b):
        for k in range(8):
            sl = pl.ds(k * 16, 16)
            rbufs[b].at[sl][...] = row_v.at[ii, sl][...]

    def _gissue(b):
        return pltpu.async_copy(xs_hbm.at[rbufs[b]], rows[b], gsem)

    pltpu.sync_copy(row_hbm.at[c, s], row_v)

    @pl.loop(0, 5)
    def _(t):
        r0 = s * 640 + t * 128
        pltpu.sync_copy(xs_hbm.at[pl.ds(c * N_PAD + r0, 128)],
                        acc_sh.at[pl.ds(r0, 128)])

    _fill_rbuf(0, 0)
    _gissue(0).wait()
    plsc.subcore_barrier()

    def _body(ii, b, do_next):
        bn = (b + 1) % 2
        de = pltpu.async_copy(ew_hbm.at[s, ii], ewb, dsem)
        dc = pltpu.async_copy(col_hbm.at[s, ii], cbuf, dsem)
        if do_next:
            _fill_rbuf(ii + 1, bn)
            d = _gissue(bn)
        de.wait()
        z16 = jnp.zeros((16,), jnp.int32)
        rowb = rows[b]

        @pl.loop(0, 128, unroll=4)
        def _(j):
            sv = plsc.load_gather(ewb, [z16 + j])
            for k in range(8):
                sl = (j, pl.ds(k * 16, 16))
                rowb.at[sl][...] = rowb.at[sl][...] * sv

        if do_next:
            d.wait()
        dc.wait()
        pltpu.sync_copy(rowb, acc_sh.at[cbuf], add=True)

    @pl.loop(0, 78, step=2)
    def _(i):
        for b in range(2):
            _body(i + b, b, True)

    _body(78, 0, True)
    _body(79, 1, False)
    plsc.subcore_barrier()

    @pl.loop(0, 5)
    def _(t):
        r0 = s * 640 + t * 128
        pltpu.sync_copy(acc_sh.at[pl.ds(r0, 128)],
                        out_hbm.at[pl.ds(c * N_PAD + r0, 128)])


@functools.partial(
    pl.kernel,
    out_type=jax.ShapeDtypeStruct((2 * B_PAIRS, 256), F32),
    mesh=_mesh,
    scratch_types=[
        pltpu.VMEM((128,), jnp.int32),
        pltpu.VMEM((128,), jnp.int32),
        pltpu.VMEM((128, 256), F32),
        pltpu.SemaphoreType.DMA,
        pltpu.SemaphoreType.DMA,
    ],
)
def _sc_gather(table_hbm, idx_hbm, out_hbm, iv0, iv1, rows_v, sem, dsem):
    c = lax.axis_index("c")
    s = lax.axis_index("s")
    ivs = (iv0, iv1)
    base = (s * 2 + c) * 512

    def _body(i, b, do_next):
        off = base + i * 128
        if do_next:
            d = pltpu.async_copy(idx_hbm.at[pl.ds(off + 128, 128)],
                                 ivs[(b + 1) % 2], dsem)
        pltpu.async_copy(table_hbm.at[ivs[b]], rows_v, sem).wait()
        pltpu.sync_copy(rows_v, out_hbm.at[pl.ds(off, 128)])
        if do_next:
            d.wait()

    pltpu.sync_copy(idx_hbm.at[pl.ds(base, 128)], ivs[0])
    _body(0, 0, True)
    _body(1, 1, True)
    _body(2, 0, True)
    _body(3, 1, False)


def kernel(Proteins, Drugs, edge_index, protein_index, drug_index, edge_weight,
           pW0, pb0, pW1, pb1, dW0, db0, dW1, db1, gW, gb, vW, vb, qW, qb,
           h_mat, h_bias, decW0, decb0, decW1, decb1, outW, outb, att_param):
    i32 = jnp.int32
    npad = NE_PAD - NE
    row = jnp.concatenate([edge_index[0], jnp.zeros((npad,), i32)])
    col = jnp.concatenate([edge_index[1], jnp.zeros((npad,), i32)])
    ew = jnp.concatenate([edge_weight, jnp.zeros((npad,), F32)])
    ew16 = jnp.broadcast_to(ew[:, None], (NE_PAD, 16))
    zeros16 = jnp.zeros((10240, 16), F32)

    deg_flat = _sc_deg(col.reshape(32, 40, 128), ew16, zeros16)
    dega = deg_flat[:N_NODES, 0:1]
    degb = deg_flat[10240:10240 + N_NODES, 0:1]

    nodes_p, x_p = _encoder(Proteins, pW0.T, pb0[None, :], pW1.T, pb1[None, :],
                            gW.T, 1000)
    nodes_d, x_d = _encoder(Drugs, dW0.T, db0[None, :], dW1.T, db1[None, :],
                            gW.T, 1000)
    nodes = jnp.concatenate([nodes_p, nodes_d], axis=0)
    x = jnp.concatenate([x_p, x_d], axis=0)

    idx_all = jnp.concatenate([protein_index, drug_index])
    g0 = _sc_gather(nodes, idx_all)
    p0, d0 = g0[:B_PAIRS], g0[B_PAIRS:]

    xs0, xs1, dis = _prep(x, dega, degb)
    zpad = jnp.zeros((N_PAD - N_NODES, 128), F32)
    xs_flat = jnp.concatenate([xs0, zpad, xs1, zpad], axis=0)
    rowp = jnp.stack([row, row + N_PAD]).reshape(2, 16, 80, 128)
    acc_flat = _sc_scatter(xs_flat, rowp, col.reshape(16, 80, 128),
                           ew.reshape(16, 80, 128))
    nodes2 = _post(acc_flat[:N_NODES], acc_flat[N_PAD:N_PAD + N_NODES], dis,
                   gb[None, :])

    g2 = _sc_gather(nodes2, idx_all)
    p2, d2 = g2[:B_PAIRS], g2[B_PAIRS:]

    hm = h_mat[0, :, 0, :].T
    hb = h_bias[0, :, 0, 0][None, :]
    pool = (jnp.arange(768)[:, None] // 3 == jnp.arange(256)[None, :]).astype(F32)
    attp = att_param[:, 0, 0][None, :]

    out = _ban(d2, p2, p0, d0, vW.T, vb[None, :], qW.T, qb[None, :], hm, hb,
               pool, attp, decW0.T, decb0[None, :], decW1.T, decb1[None, :],
               outW.T, outb[None, :])
    return out.reshape(-1)


# speedup vs baseline: 1.0966x; 1.0273x over previous
import dataclasses
import functools

import jax
import jax.numpy as jnp
from jax import lax
from jax.experimental import pallas as pl
from jax.experimental.pallas import tpu as pltpu
from jax.experimental.pallas import tpu_sc as plsc

F32 = jnp.float32
BN = (1.0 + 1e-5) ** -0.5

N_NODES = 10000
NE = 160000
NE_PAD = 163840
B_PAIRS = 8192
N_PAD = 10240

_mesh = plsc.VectorSubcoreMesh(core_axis_name="c", subcore_axis_name="s")

_sc_params = pltpu.CompilerParams()
if "needs_layout_passes" in pltpu.CompilerParams.__dataclass_fields__:
    _sc_params = dataclasses.replace(_sc_params, needs_layout_passes=False)


def _dot(a, b):
    return jnp.dot(a, b, preferred_element_type=F32)


def _ln(x):
    m = jnp.mean(x, axis=-1, keepdims=True)
    v = jnp.mean((x - m) ** 2, axis=-1, keepdims=True)
    return (x - m) / jnp.sqrt(v + 1e-5)


def _enc_body(x_ref, w0_ref, b0_ref, w1_ref, b1_ref, gw_ref, nodes_ref, x_out_ref):
    z = _dot(x_ref[...], w0_ref[...]) + b0_ref[...]
    a = _ln(BN * jnp.maximum(z, 0.0))
    z = _dot(a, w1_ref[...]) + b1_ref[...]
    b = _ln(BN * jnp.maximum(z, 0.0))
    node = _ln(BN * b)
    nodes_ref[...] = node
    x_out_ref[...] = _dot(node, gw_ref[...])


def _encoder(x, w0t, b0, w1t, b1, gwt, block):
    n, din = x.shape
    dmid = w0t.shape[1]
    grid = (n // block,)
    full = lambda shape: pl.BlockSpec(shape, lambda i: (0, 0))
    return pl.pallas_call(
        _enc_body,
        grid=grid,
        in_specs=[
            pl.BlockSpec((block, din), lambda i: (i, 0)),
            full((din, dmid)),
            full((1, dmid)),
            full((dmid, 256)),
            full((1, 256)),
            full((256, 256)),
        ],
        out_specs=[
            pl.BlockSpec((block, 256), lambda i: (i, 0)),
            pl.BlockSpec((block, 256), lambda i: (i, 0)),
        ],
        out_shape=[
            jax.ShapeDtypeStruct((n, 256), F32),
            jax.ShapeDtypeStruct((n, 256), F32),
        ],
    )(x, w0t, b0, w1t, b1, gwt)


def _prep_body(x_ref, da_ref, db_ref, xs0_ref, xs1_ref, dis_ref):
    deg = da_ref[...] + db_ref[...] + 1.0
    dis = lax.rsqrt(deg)
    xs = x_ref[...] * dis
    xs0_ref[...] = xs[:, :128]
    xs1_ref[...] = xs[:, 128:]
    dis_ref[...] = dis


def _prep(x, dega, degb):
    block = 1000
    grid = (N_NODES // block,)
    return pl.pallas_call(
        _prep_body,
        grid=grid,
        in_specs=[
            pl.BlockSpec((block, 256), lambda i: (i, 0)),
            pl.BlockSpec((block, 1), lambda i: (i, 0)),
            pl.BlockSpec((block, 1), lambda i: (i, 0)),
        ],
        out_specs=[
            pl.BlockSpec((block, 128), lambda i: (i, 0)),
            pl.BlockSpec((block, 128), lambda i: (i, 0)),
            pl.BlockSpec((block, 1), lambda i: (i, 0)),
        ],
        out_shape=[
            jax.ShapeDtypeStruct((N_NODES, 128), F32),
            jax.ShapeDtypeStruct((N_NODES, 128), F32),
            jax.ShapeDtypeStruct((N_NODES, 1), F32),
        ],
    )(x, dega, degb)


def _post_body(a0_ref, a1_ref, dis_ref, gb_ref, out_ref):
    agg = jnp.concatenate([a0_ref[...], a1_ref[...]], axis=1)
    agg = agg * dis_ref[...] + gb_ref[...]
    e = jnp.where(agg > 0.0, agg, jnp.exp(agg) - 1.0)
    out_ref[...] = _ln(BN * e)


def _post(acc0, acc1, dis, gb):
    block = 1000
    grid = (N_NODES // block,)
    return pl.pallas_call(
        _post_body,
        grid=grid,
        in_specs=[
            pl.BlockSpec((block, 128), lambda i: (i, 0)),
            pl.BlockSpec((block, 128), lambda i: (i, 0)),
            pl.BlockSpec((block, 1), lambda i: (i, 0)),
            pl.BlockSpec((1, 256), lambda i: (0, 0)),
        ],
        out_specs=pl.BlockSpec((block, 256), lambda i: (i, 0)),
        out_shape=jax.ShapeDtypeStruct((N_NODES, 256), F32),
    )(acc0, acc1, dis, gb)


def _ban_body(d2_ref, p2_ref, p0_ref, d0_ref, vw_ref, vb_ref, qw_ref, qb_ref,
              hm_ref, hb_ref, pool_ref, attp_ref, w0_ref, b0_ref, w1_ref,
              b1_ref, wo_ref, bo_ref, out_ref):
    v = jnp.maximum(_dot(d2_ref[...], vw_ref[...]) + vb_ref[...], 0.0)
    q = jnp.maximum(_dot(p2_ref[...], qw_ref[...]) + qb_ref[...], 0.0)
    p = v * q
    s = _dot(p, hm_ref[...]) + hb_ref[...]
    ssum = s[:, 0:1] + s[:, 1:2]
    pooled = _dot(p, pool_ref[...])
    logits = ssum * pooled
    ap = attp_ref[...]
    e = jnp.exp(ap - jnp.max(ap))
    a = e / jnp.sum(e)
    pair = (a[0:1, 0:1] * (BN * logits) + a[0:1, 1:2] * p0_ref[...]
            + a[0:1, 2:3] * d0_ref[...])
    pair = BN * jnp.maximum(_dot(pair, w0_ref[...]) + b0_ref[...], 0.0)
    pair = BN * jnp.maximum(_dot(pair, w1_ref[...]) + b1_ref[...], 0.0)
    z = _dot(pair, wo_ref[...]) + bo_ref[...]
    out_ref[...] = 1.0 / (1.0 + jnp.exp(-z))


def _ban(d2, p2, p0, d0, vwt, vb, qwt, qb, hm, hb, pool, attp, w0t, b0, w1t,
         b1, wot, bo):
    block = 1024
    grid = (B_PAIRS // block,)
    rowspec = pl.BlockSpec((block, 256), lambda i: (i, 0))
    full = lambda shape: pl.BlockSpec(shape, lambda i: (0, 0))
    return pl.pallas_call(
        _ban_body,
        grid=grid,
        in_specs=[
            rowspec, rowspec, rowspec, rowspec,
            full((256, 768)), full((1, 768)),
            full((256, 768)), full((1, 768)),
            full((768, 2)), full((1, 2)),
            full((768, 256)), full((1, 3)),
            full((256, 256)), full((1, 256)),
            full((256, 128)), full((1, 128)),
            full((128, 1)), full((1, 1)),
        ],
        out_specs=pl.BlockSpec((block, 1), lambda i: (i, 0)),
        out_shape=jax.ShapeDtypeStruct((B_PAIRS, 1), F32),
    )(d2, p2, p0, d0, vwt, vb, qwt, qb, hm, hb, pool, attp, w0t, b0, w1t, b1,
      wot, bo)


@functools.partial(
    pl.kernel,
    out_type=jax.ShapeDtypeStruct((20480, 16), F32),
    mesh=_mesh,
    scratch_types=[
        pltpu.VMEM((128,), jnp.int32),
        pltpu.VMEM((128,), jnp.int32),
        pltpu.VMEM((128, 16), F32),
        pltpu.VMEM((128, 16), F32),
        pltpu.VMEM_SHARED((10240, 16), F32),
        pltpu.SemaphoreType.DMA,
    ],
)
def _sc_deg(col_hbm, ew16_hbm, zeros_hbm, out_hbm, cv0, cv1, vv0, vv1,
            acc_sh, dsem):
    c = lax.axis_index("c")
    s = lax.axis_index("s")
    cols = (cv0, cv1)
    vals = (vv0, vv1)
    w = c * 16 + s
    pltpu.sync_copy(zeros_hbm.at[pl.ds(s * 640, 640)],
                    acc_sh.at[pl.ds(s * 640, 640)])

    def _body(i, b, do_next):
        bn = (b + 1) % 2
        if do_next:
            d1 = pltpu.async_copy(col_hbm.at[w, i + 1], cols[bn], dsem)
            d2 = pltpu.async_copy(
                ew16_hbm.at[pl.ds((w * 40 + i + 1) * 128, 128)], vals[bn],
                dsem)
        pltpu.sync_copy(vals[b], acc_sh.at[cols[b]], add=True)
        if do_next:
            d1.wait()
            d2.wait()

    pltpu.sync_copy(col_hbm.at[w, 0], cols[0])
    pltpu.sync_copy(ew16_hbm.at[pl.ds(w * 40 * 128, 128)], vals[0])
    plsc.subcore_barrier()

    @pl.loop(0, 38, step=2)
    def _(i):
        for b in range(2):
            _body(i + b, b, True)

    _body(38, 0, True)
    _body(39, 1, False)
    plsc.subcore_barrier()
    pltpu.sync_copy(acc_sh.at[pl.ds(s * 640, 640)],
                    out_hbm.at[pl.ds(c * 10240 + s * 640, 640)])


@functools.partial(
    pl.kernel,
    out_type=jax.ShapeDtypeStruct((2 * N_PAD, 128), F32),
    mesh=_mesh,
    scratch_types=[
        pltpu.VMEM((80, 128), jnp.int32),
        pltpu.VMEM((128, 128), F32),
        pltpu.VMEM((128, 128), F32),
        pltpu.VMEM((128,), jnp.int32),
        pltpu.VMEM((128,), jnp.int32),
        pltpu.VMEM((128,), jnp.int32),
        pltpu.VMEM((128,), F32),
        pltpu.VMEM_SHARED((N_PAD, 128), F32),
        pltpu.SemaphoreType.DMA,
        pltpu.SemaphoreType.DMA,
        pltpu.SemaphoreType.DMA,
    ],
    compiler_params=_sc_params,
)
def _sc_scatter(xs_hbm, row_hbm, col_hbm, ew_hbm, out_hbm, row_v, rw0, rw1,
                rbuf0, rbuf1, cbuf, ewb, acc_sh, gsem, dsem, ssem):
    c = lax.axis_index("c")
    s = lax.axis_index("s")
    rows = (rw0, rw1)
    rbufs = (rbuf0, rbuf1)

    def _fill_rbuf(ii, b):
        for k in range(8):
            sl = pl.ds(k * 16, 16)
            rbufs[b].at[sl][...] = row_v.at[ii, sl][...]

    def _gissue(b):
        return pltpu.async_copy(xs_hbm.at[rbufs[b]], rows[b], gsem)

    pltpu.sync_copy(row_hbm.at[c, s], row_v)

    @pl.loop(0, 5)
    def _(t):
        r0 = s * 640 + t * 128
        pltpu.sync_copy(xs_hbm.at[pl.ds(c * N_PAD + r0, 128)],
                        acc_sh.at[pl.ds(r0, 128)])

    _fill_rbuf(0, 0)
    _gissue(0).wait()
    plsc.subcore_barrier()

    def _body(ii, b, do_next):
        bn = (b + 1) % 2
        de = pltpu.async_copy(ew_hbm.at[s, ii], ewb, dsem)
        dc = pltpu.async_copy(col_hbm.at[s, ii], cbuf, dsem)
        if do_next:
            _fill_rbuf(ii + 1, bn)
            d = _gissue(bn)
        de.wait()
        z16 = jnp.zeros((16,), jnp.int32)
        rowb = rows[b]

        @pl.loop(0, 128, unroll=4)
        def _(j):
            sv = plsc.load_gather(ewb, [z16 + j])
            for k in range(8):
                sl = (j, pl.ds(k * 16, 16))
                rowb.at[sl][...] = rowb.at[sl][...] * sv

        dc.wait()
        ds = pltpu.async_copy(rowb, acc_sh.at[cbuf], ssem, add=True)
        if do_next:
            d.wait()
        ds.wait()

    @pl.loop(0, 78, step=2)
    def _(i):
        for b in range(2):
            _body(i + b, b, True)

    _body(78, 0, True)
    _body(79, 1, False)
    plsc.subcore_barrier()

    @pl.loop(0, 5)
    def _(t):
        r0 = s * 640 + t * 128
        pltpu.sync_copy(acc_sh.at[pl.ds(r0, 128)],
                        out_hbm.at[pl.ds(c * N_PAD + r0, 128)])


@functools.partial(
    pl.kernel,
    out_type=jax.ShapeDtypeStruct((2 * B_PAIRS, 256), F32),
    mesh=_mesh,
    scratch_types=[
        pltpu.VMEM((128,), jnp.int32),
        pltpu.VMEM((128,), jnp.int32),
        pltpu.VMEM((128, 256), F32),
        pltpu.SemaphoreType.DMA,
        pltpu.SemaphoreType.DMA,
    ],
)
def _sc_gather(table_hbm, idx_hbm, out_hbm, iv0, iv1, rows_v, sem, dsem):
    c = lax.axis_index("c")
    s = lax.axis_index("s")
    ivs = (iv0, iv1)
    base = (s * 2 + c) * 512

    def _body(i, b, do_next):
        off = base + i * 128
        if do_next:
            d = pltpu.async_copy(idx_hbm.at[pl.ds(off + 128, 128)],
                                 ivs[(b + 1) % 2], dsem)
        pltpu.async_copy(table_hbm.at[ivs[b]], rows_v, sem).wait()
        pltpu.sync_copy(rows_v, out_hbm.at[pl.ds(off, 128)])
        if do_next:
            d.wait()

    pltpu.sync_copy(idx_hbm.at[pl.ds(base, 128)], ivs[0])
    _body(0, 0, True)
    _body(1, 1, True)
    _body(2, 0, True)
    _body(3, 1, False)


def kernel(Proteins, Drugs, edge_index, protein_index, drug_index, edge_weight,
           pW0, pb0, pW1, pb1, dW0, db0, dW1, db1, gW, gb, vW, vb, qW, qb,
           h_mat, h_bias, decW0, decb0, decW1, decb1, outW, outb, att_param):
    i32 = jnp.int32
    npad = NE_PAD - NE
    row = jnp.concatenate([edge_index[0], jnp.zeros((npad,), i32)])
    col = jnp.concatenate([edge_index[1], jnp.zeros((npad,), i32)])
    ew = jnp.concatenate([edge_weight, jnp.zeros((npad,), F32)])
    ew16 = jnp.broadcast_to(ew[:, None], (NE_PAD, 16))
    zeros16 = jnp.zeros((10240, 16), F32)

    deg_flat = _sc_deg(col.reshape(32, 40, 128), ew16, zeros16)
    dega = deg_flat[:N_NODES, 0:1]
    degb = deg_flat[10240:10240 + N_NODES, 0:1]

    nodes_p, x_p = _encoder(Proteins, pW0.T, pb0[None, :], pW1.T, pb1[None, :],
                            gW.T, 1000)
    nodes_d, x_d = _encoder(Drugs, dW0.T, db0[None, :], dW1.T, db1[None, :],
                            gW.T, 1000)
    nodes = jnp.concatenate([nodes_p, nodes_d], axis=0)
    x = jnp.concatenate([x_p, x_d], axis=0)

    idx_all = jnp.concatenate([protein_index, drug_index])
    g0 = _sc_gather(nodes, idx_all)
    p0, d0 = g0[:B_PAIRS], g0[B_PAIRS:]

    xs0, xs1, dis = _prep(x, dega, degb)
    zpad = jnp.zeros((N_PAD - N_NODES, 128), F32)
    xs_flat = jnp.concatenate([xs0, zpad, xs1, zpad], axis=0)
    rowp = jnp.stack([row, row + N_PAD]).reshape(2, 16, 80, 128)
    acc_flat = _sc_scatter(xs_flat, rowp, col.reshape(16, 80, 128),
                           ew.reshape(16, 80, 128))
    nodes2 = _post(acc_flat[:N_NODES], acc_flat[N_PAD:N_PAD + N_NODES], dis,
                   gb[None, :])

    g2 = _sc_gather(nodes2, idx_all)
    p2, d2 = g2[:B_PAIRS], g2[B_PAIRS:]

    hm = h_mat[0, :, 0, :].T
    hb = h_bias[0, :, 0, 0][None, :]
    pool = (jnp.arange(768)[:, None] // 3 == jnp.arange(256)[None, :]).astype(F32)
    attp = att_param[:, 0, 0][None, :]

    out = _ban(d2, p2, p0, d0, vW.T, vb[None, :], qW.T, qb[None, :], hm, hb,
               pool, attp, decW0.T, decb0[None, :], decW1.T, decb1[None, :],
               outW.T, outb[None, :])
    return out.reshape(-1)
